# Initial kernel scaffold; baseline (speedup 1.0000x reference)
import jax
import jax.numpy as jnp
from jax.experimental import pallas as pl


def kernel(x, W_base, b_base, W_router, lora_A, lora_B):
    raise NotImplementedError("write your pallas kernel here")


# speedup vs baseline: 10.4166x; 10.4166x over previous
import functools

import jax
import jax.numpy as jnp
from jax.experimental import pallas as pl
from jax.experimental.pallas import tpu as pltpu

D_MODEL = 1024
NUM_EXPERTS = 8
TOP_K = 2
R = 16
SCALING = 32.0 / 16.0

ER = NUM_EXPERTS * R
BLOCK_N = 1024


def _fused_kernel(x_ref, w_ref, b_ref, wr_ref, a_ref, bb_ref, o_ref):
    xb = x_ref[...]
    base = jax.lax.dot_general(
        xb, w_ref[...], (((1,), (1,)), ((), ())),
        preferred_element_type=jnp.float32)
    base = base + b_ref[...]

    logits = jax.lax.dot_general(
        xb, wr_ref[...], (((1,), (1,)), ((), ())),
        preferred_element_type=jnp.float32)
    bn = logits.shape[0]
    lane = jax.lax.broadcasted_iota(jnp.int32, (bn, ER), 1)
    neg = jnp.float32(-jnp.inf)
    logits = jnp.where(lane < NUM_EXPERTS, logits, neg)

    m1 = jnp.max(logits, axis=1, keepdims=True)
    i1 = jnp.min(jnp.where(logits == m1, lane, ER), axis=1, keepdims=True)
    logits2 = jnp.where(lane == i1, neg, logits)
    m2 = jnp.max(logits2, axis=1, keepdims=True)
    i2 = jnp.min(jnp.where(logits2 == m2, lane, ER), axis=1, keepdims=True)
    eb = jnp.exp(m2 - m1)
    denom = 1.0 + eb
    g1 = 1.0 / denom
    g2 = eb / denom
    eidx = lane // R
    gates = jnp.where(eidx == i1, g1, 0.0) + jnp.where(eidx == i2, g2, 0.0)

    hidden = jax.lax.dot_general(
        xb, a_ref[...], (((1,), (0,)), ((), ())),
        preferred_element_type=jnp.float32)
    hidden = hidden * (gates * SCALING)
    delta = jax.lax.dot_general(
        hidden, bb_ref[...], (((1,), (0,)), ((), ())),
        preferred_element_type=jnp.float32)
    o_ref[...] = base + delta


@functools.partial(jax.jit, static_argnames=())
def kernel(x, W_base, b_base, W_router, lora_A, lora_B):
    n, d = x.shape
    A_all = lora_A.transpose(1, 0, 2).reshape(d, ER)
    B_all = lora_B.reshape(ER, d)
    Wr_pad = jnp.zeros((ER, d), W_router.dtype).at[:NUM_EXPERTS].set(W_router)
    b2 = b_base.reshape(1, d)

    grid = (n // BLOCK_N,)
    out = pl.pallas_call(
        _fused_kernel,
        grid=grid,
        in_specs=[
            pl.BlockSpec((BLOCK_N, d), lambda i: (i, 0)),
            pl.BlockSpec((d, d), lambda i: (0, 0)),
            pl.BlockSpec((1, d), lambda i: (0, 0)),
            pl.BlockSpec((ER, d), lambda i: (0, 0)),
            pl.BlockSpec((d, ER), lambda i: (0, 0)),
            pl.BlockSpec((ER, d), lambda i: (0, 0)),
        ],
        out_specs=pl.BlockSpec((BLOCK_N, d), lambda i: (i, 0)),
        out_shape=jax.ShapeDtypeStruct((n, d), x.dtype),
        compiler_params=pltpu.CompilerParams(
            dimension_semantics=("arbitrary",)),
    )(x, W_base, b2, Wr_pad, A_all, B_all)
    return out
